# Initial kernel scaffold; baseline (speedup 1.0000x reference)
import jax
import jax.numpy as jnp
from jax.experimental import pallas as pl


def kernel(x, embedding):
    raise NotImplementedError("write your pallas kernel here")


# speedup vs baseline: 6.3479x; 6.3479x over previous
import functools

import jax
import jax.numpy as jnp
from jax import lax
from jax.experimental import pallas as pl
from jax.experimental.pallas import tpu as pltpu
from jax.experimental.pallas import tpu_sc as plsc

DIM = 128
CHUNK = 128


@functools.cache
def _build(n_rows, dim):
    info = plsc.get_sparse_core_info()
    nc, ns = info.num_cores, info.num_subcores
    nw = nc * ns
    n_chunks = n_rows // (nw * CHUNK)
    assert n_chunks * nw * CHUNK == n_rows

    mesh = plsc.VectorSubcoreMesh(core_axis_name="c", subcore_axis_name="s")

    @functools.partial(
        pl.kernel,
        mesh=mesh,
        out_type=jax.ShapeDtypeStruct((nw, n_chunks, CHUNK, dim), jnp.float32),
        scratch_types=[
            pltpu.VMEM((n_chunks, CHUNK), jnp.int32),
            pltpu.VMEM((CHUNK, dim), jnp.float32),
            pltpu.SemaphoreType.DMA,
        ],
    )
    def gather_kernel(idx_hbm, table_hbm, out_hbm, idx_v, rows_v, gsem):
        wid = lax.axis_index("s") * nc + lax.axis_index("c")
        pltpu.sync_copy(idx_hbm.at[wid], idx_v)

        def body(c, carry):
            pltpu.async_copy(table_hbm.at[idx_v.at[c]], rows_v, gsem).wait()
            pltpu.sync_copy(rows_v, out_hbm.at[wid, c])
            return carry

        lax.fori_loop(0, n_chunks, body, 0, unroll=False)

    return gather_kernel, nw, n_chunks


def kernel(x, embedding):
    b, h = x.shape
    dim = embedding.shape[1]
    n_rows = b * h
    gather_kernel, nw, n_chunks = _build(n_rows, dim)
    idx = x.reshape(nw, n_chunks, CHUNK)
    out = gather_kernel(idx, embedding)
    return out.reshape(b, h, dim)


# speedup vs baseline: 8.4774x; 1.3355x over previous
import functools

import jax
import jax.numpy as jnp
from jax import lax
from jax.experimental import pallas as pl
from jax.experimental.pallas import tpu as pltpu
from jax.experimental.pallas import tpu_sc as plsc

DIM = 128
CHUNK = 128
NBUF = 2


@functools.cache
def _build(n_rows, dim):
    info = plsc.get_sparse_core_info()
    nc, ns = info.num_cores, info.num_subcores
    nw = nc * ns
    n_chunks = n_rows // (nw * CHUNK)
    assert n_chunks * nw * CHUNK == n_rows and n_chunks % NBUF == 0

    mesh = plsc.VectorSubcoreMesh(core_axis_name="c", subcore_axis_name="s")

    @functools.partial(
        pl.kernel,
        mesh=mesh,
        out_type=jax.ShapeDtypeStruct((nw, n_chunks, CHUNK, dim), jnp.float32),
        scratch_types=[
            pltpu.VMEM((n_chunks, CHUNK), jnp.int32),
            pltpu.VMEM((NBUF, CHUNK, dim), jnp.float32),
        ]
        + [pltpu.SemaphoreType.DMA] * (2 * NBUF),
    )
    def gather_kernel(idx_hbm, table_hbm, out_hbm, idx_v, rows_v, *sems):
        gsem, wsem = sems[:NBUF], sems[NBUF:]
        wid = lax.axis_index("s") * nc + lax.axis_index("c")
        pltpu.sync_copy(idx_hbm.at[wid], idx_v)

        def gfire(c, b):
            pltpu.async_copy(table_hbm.at[idx_v.at[c]], rows_v.at[b], gsem[b])

        def gwait(b):
            pltpu.make_async_copy(
                table_hbm.at[idx_v.at[0]], rows_v.at[b], gsem[b]
            ).wait()

        def wfire(c, b):
            pltpu.async_copy(rows_v.at[b], out_hbm.at[wid, c], wsem[b])

        def wwait(b):
            pltpu.make_async_copy(rows_v.at[b], out_hbm.at[wid, 0], wsem[b]).wait()

        for b in range(NBUF):
            gfire(b, b)

        def body(j, carry):
            for b in range(NBUF):
                c = j * NBUF + b
                gwait(b)
                wfire(c, b)

                @pl.when(c + NBUF < n_chunks)
                def _():
                    wwait(b)
                    gfire(c + NBUF, b)

            return carry

        lax.fori_loop(0, n_chunks // NBUF, body, 0, unroll=False)
        for b in range(NBUF):
            wwait(b)

    return gather_kernel, nw, n_chunks


def kernel(x, embedding):
    b, h = x.shape
    dim = embedding.shape[1]
    n_rows = b * h
    gather_kernel, nw, n_chunks = _build(n_rows, dim)
    idx = x.reshape(nw, n_chunks, CHUNK)
    out = gather_kernel(idx, embedding)
    return out.reshape(b, h, dim)
